# baseline (speedup 1.0000x reference)
import functools
import jax
import jax.numpy as jnp
from jax import lax
from jax.experimental import pallas as pl
from jax.experimental.pallas import tpu as pltpu
from jax.experimental.pallas import tpu_sc as plsc

HIDDEN = 1024
E = 8
TOP_K = 2
D_FF = 512
N_GROUP = 4
TOPK_GROUP = 2
ROUTED_SCALE = 2.5
T = 2048
SHARED_FF = 1024

NW = 32
TPW = T // NW
BT = 256
LOG_BT = 8
NB = 4096 // BT + 8
NBP = 32
NP = 4096 + 8 * BT
TBA = 512

@functools.cache
def _mesh():
    return plsc.VectorSubcoreMesh(core_axis_name="c", subcore_axis_name="s",
                                  num_cores=2, num_subcores=16)


def _silu(v):
    return v * (1.0 / (1.0 + jnp.exp(-v)))


def _a1_body(x_ref, gate_ref, lt_ref):
    lt_ref[:] = lax.dot_general(gate_ref[:], x_ref[:], (((0,), (1,)), ((), ())),
                                preferred_element_type=jnp.float32)


def _stage_a1(x, gate_w):
    return pl.pallas_call(
        _a1_body,
        grid=(T // TBA,),
        in_specs=[
            pl.BlockSpec((TBA, HIDDEN), lambda i: (i, 0)),
            pl.BlockSpec((HIDDEN, E), lambda i: (0, 0)),
        ],
        out_specs=pl.BlockSpec((E, TBA), lambda i: (0, i)),
        out_shape=jax.ShapeDtypeStruct((E, T), jnp.float32),
        compiler_params=pltpu.CompilerParams(
            dimension_semantics=("arbitrary",)),
    )(x, gate_w)


def _a2_body(x_ref, swg_ref, swu_ref, swd_ref, sh_ref, swg_bf, swu_bf,
             swd_bf):
    @pl.when(pl.program_id(0) == 0)
    def _():
        swg_bf[:] = swg_ref[:].astype(jnp.bfloat16)
        swu_bf[:] = swu_ref[:].astype(jnp.bfloat16)
        swd_bf[:] = swd_ref[:].astype(jnp.bfloat16)

    xbf = x_ref[:].astype(jnp.bfloat16)
    sg = jnp.dot(xbf, swg_bf[:], preferred_element_type=jnp.float32)
    su = jnp.dot(xbf, swu_bf[:], preferred_element_type=jnp.float32)
    hs = (_silu(sg) * su).astype(jnp.bfloat16)
    sh_ref[:] = jnp.dot(hs, swd_bf[:], preferred_element_type=jnp.float32)


def _stage_a2(x, sw_gate, sw_up, sw_down):
    full = lambda shape: pl.BlockSpec(shape, lambda i: (0,) * len(shape))
    return pl.pallas_call(
        _a2_body,
        grid=(T // TBA,),
        in_specs=[
            pl.BlockSpec((TBA, HIDDEN), lambda i: (i, 0)),
            full((HIDDEN, SHARED_FF)),
            full((HIDDEN, SHARED_FF)),
            full((SHARED_FF, HIDDEN)),
        ],
        out_specs=pl.BlockSpec((TBA, HIDDEN), lambda i: (i, 0)),
        out_shape=jax.ShapeDtypeStruct((T, HIDDEN), jnp.float32),
        scratch_shapes=[
            pltpu.VMEM((HIDDEN, SHARED_FF), jnp.bfloat16),
            pltpu.VMEM((HIDDEN, SHARED_FF), jnp.bfloat16),
            pltpu.VMEM((SHARED_FF, HIDDEN), jnp.bfloat16),
        ],
        compiler_params=pltpu.CompilerParams(
            dimension_semantics=("arbitrary",)),
    )(x, sw_gate, sw_up, sw_down)


def _routing_batch(le):
    one = jnp.ones((16,), jnp.int32)
    zi = jnp.zeros((16,), jnp.int32)
    zf = jnp.zeros((16,), jnp.float32)
    gc = [jnp.maximum(le[2 * j], le[2 * j + 1]) for j in range(N_GROUP)]
    gsel = []
    for j in range(N_GROUP):
        beaten = zi
        for j2 in range(N_GROUP):
            if j2 == j:
                continue
            b = (gc[j2] >= gc[j]) if j2 < j else (gc[j2] > gc[j])
            beaten = beaten + jnp.where(b, one, zi)
        gsel.append(beaten < TOPK_GROUP)
    valid = [gsel[e // 2] for e in range(E)]
    esel = []
    for e in range(E):
        beaten = zi
        for e2 in range(E):
            if e2 == e:
                continue
            b = (le[e2] >= le[e]) if e2 < e else (le[e2] > le[e])
            beaten = beaten + jnp.where(valid[e2] & b, one, zi)
        esel.append(valid[e] & (beaten < TOP_K))
    big = jnp.full((16,), 99, jnp.int32)
    neg1 = jnp.full((16,), -1, jnp.int32)
    ev = [jnp.full((16,), e, jnp.int32) for e in range(E)]
    idx0 = functools.reduce(jnp.minimum,
                            [jnp.where(esel[e], ev[e], big) for e in range(E)])
    idx1 = functools.reduce(jnp.maximum,
                            [jnp.where(esel[e], ev[e], neg1) for e in range(E)])
    neginf = jnp.full((16,), -1e30, jnp.float32)
    neg20 = jnp.full((16,), -20.0, jnp.float32)
    m = functools.reduce(
        jnp.maximum, [jnp.where(esel[e], le[e], neginf) for e in range(E)])
    we = [jnp.where(esel[e], jnp.exp(jnp.where(esel[e], le[e] - m, neg20)),
                    zf) for e in range(E)]
    wsum = functools.reduce(jnp.add, we)
    winv = jnp.full((16,), ROUTED_SCALE, jnp.float32) / wsum
    w0 = functools.reduce(
        jnp.add,
        [jnp.where(idx0 == ev[e], we[e], zf) for e in range(E)]) * winv
    w1 = functools.reduce(
        jnp.add,
        [jnp.where(idx1 == ev[e], we[e], zf) for e in range(E)]) * winv
    return idx0, idx1, w0, w1, esel


def _b1_body(lt_hbm, idx0_hbm, idx1_hbm, w0_hbm, w1_hbm, counts_hbm,
             lt_v, i0_v, i1_v, w0_v, w1_v, cnt_v):
    wid = lax.axis_index("s") * 2 + lax.axis_index("c")
    base = wid * TPW
    for e in range(E):
        pltpu.sync_copy(lt_hbm.at[e, pl.ds(base, TPW)], lt_v.at[e])
    ii = lax.broadcasted_iota(jnp.int32, (16,), 0)
    zi = jnp.zeros((16,), jnp.int32)
    cnt = zi
    for b in range(TPW // 16):
        le = [lt_v[e, pl.ds(b * 16, 16)] for e in range(E)]
        idx0, idx1, w0, w1, esel = _routing_batch(le)
        i0_v[pl.ds(b * 16, 16)] = idx0
        i1_v[pl.ds(b * 16, 16)] = idx1
        w0_v[pl.ds(b * 16, 16)] = w0
        w1_v[pl.ds(b * 16, 16)] = w1
        for e in range(E):
            pope = plsc.all_reduce_population_count(esel[e])
            cnt = cnt + jnp.where(ii == jnp.full((16,), e, jnp.int32),
                                  pope, zi)
    cnt_v[...] = cnt
    pltpu.sync_copy(i0_v, idx0_hbm.at[pl.ds(base, TPW)])
    pltpu.sync_copy(i1_v, idx1_hbm.at[pl.ds(base, TPW)])
    pltpu.sync_copy(w0_v, w0_hbm.at[pl.ds(base, TPW)])
    pltpu.sync_copy(w1_v, w1_hbm.at[pl.ds(base, TPW)])
    pltpu.sync_copy(cnt_v, counts_hbm.at[wid])


@functools.cache
def _b1():
    return functools.partial(
        pl.kernel,
        out_type=[
            jax.ShapeDtypeStruct((T,), jnp.int32),
            jax.ShapeDtypeStruct((T,), jnp.int32),
            jax.ShapeDtypeStruct((T,), jnp.float32),
            jax.ShapeDtypeStruct((T,), jnp.float32),
            jax.ShapeDtypeStruct((NW, 16), jnp.int32),
        ],
        mesh=_mesh(),
        compiler_params=pltpu.CompilerParams(needs_layout_passes=False),
        scratch_types=[
            pltpu.VMEM((E, TPW), jnp.float32),
            pltpu.VMEM((TPW,), jnp.int32),
            pltpu.VMEM((TPW,), jnp.int32),
            pltpu.VMEM((TPW,), jnp.float32),
            pltpu.VMEM((TPW,), jnp.float32),
            pltpu.VMEM((16,), jnp.int32),
        ],
    )(_b1_body)


def _splat(v, e_scalar):
    ii = lax.broadcasted_iota(jnp.int32, (16,), 0)
    s = lax.reduce_sum_p.bind(
        jnp.where(ii == jnp.full((16,), e_scalar, jnp.int32), v,
                  jnp.zeros_like(v)), axes=(0,))
    return jnp.full((16,), s, v.dtype)


def _b2_body(counts_hbm, idx0_hbm, idx1_hbm, x_hbm,
             xs_hbm, pos0_hbm, pos1_hbm, bexp_hbm, nbu_hbm,
             counts_v, i0_v, i1_v, p0_v, p1_v, bexp_v, nbu_v, xrows_v,
             pmat_v, sems, semx):
    wid = lax.axis_index("s") * 2 + lax.axis_index("c")
    base = wid * TPW
    ii = lax.broadcasted_iota(jnp.int32, (16,), 0)
    xcopy = pltpu.async_copy(x_hbm.at[pl.ds(base, TPW)], xrows_v, semx)
    pltpu.sync_copy(counts_hbm, counts_v)

    zero = jnp.zeros((16,), jnp.int32)
    wid_v = jnp.full((16,), wid, jnp.int32)
    total = zero
    pre = zero
    for w in range(NW):
        row = counts_v[w]
        total = total + row
        pre = pre + jnp.where(jnp.full((16,), w, jnp.int32) < wid_v, row,
                              zero)
    padded = lax.shift_left(
        lax.shift_right_logical(total + (BT - 1), LOG_BT), LOG_BT)
    cums = plsc.cumsum(padded)
    offp = cums - padded
    wb = offp + pre

    nbu = _splat(cums, 7)
    nbu_blocks = lax.shift_right_logical(nbu, LOG_BT)
    starts = lax.shift_right_logical(offp, LOG_BT)
    nblk = lax.shift_right_logical(padded, LOG_BT)
    laste = lax.reduce_max_p.bind(jnp.where(nblk > zero, ii, zero),
                                  axes=(0,))
    laste_v = jnp.full((16,), laste, jnp.int32)
    for v in range(NBP // 16):
        b = ii + jnp.full((16,), v * 16, jnp.int32)
        val = zero
        for e in range(E):
            s_e = _splat(starts, e)
            n_e = _splat(nblk, e)
            val = val + jnp.where((b >= s_e) & (b < s_e + n_e),
                                  jnp.full((16,), e, jnp.int32), zero)
        val = jnp.where(b >= nbu_blocks, laste_v, val)
        bexp_v[pl.ds(v * 16, 16)] = val
    nbu_v[...] = nbu_blocks

    @pl.when(wid == 0)
    def _():
        pltpu.sync_copy(bexp_v, bexp_hbm)
        pltpu.sync_copy(nbu_v, nbu_hbm)

    pltpu.sync_copy(idx0_hbm.at[pl.ds(base, TPW)], i0_v)
    pltpu.sync_copy(idx1_hbm.at[pl.ds(base, TPW)], i1_v)
    ctr = zero
    onev = jnp.ones((16,), jnp.int32)
    pend = []
    for b in range(TPW // 16):
        for k, (iv, pv) in enumerate(((i0_v, p0_v), (i1_v, p1_v))):
            idxv = iv[pl.ds(b * 16, 16)]
            posv = zero
            for e in range(E):
                e_v = jnp.full((16,), e, jnp.int32)
                msk = idxv == e_v
                mi = jnp.where(msk, onev, zero)
                cs = plsc.cumsum(mi)
                lane_base = _splat(wb + ctr, e)
                posv = posv + jnp.where(msk, lane_base + cs - onev, zero)
                pope = plsc.all_reduce_population_count(msk)
                ctr = ctr + jnp.where(ii == e_v, pope, zero)
            pv[pl.ds(b * 16, 16)] = posv
            j = 2 * b + k
            pmat_v[j] = posv
            if j == 0:
                xcopy.wait()
            if j >= 4:
                pend[j - 4].wait()
            pend.append(pltpu.async_copy(
                xrows_v.at[pl.ds(b * 16, 16)],
                xs_hbm.at[pmat_v.at[j]], sems.at[j % 4]))
    pltpu.sync_copy(p0_v, pos0_hbm.at[pl.ds(base, TPW)])
    pltpu.sync_copy(p1_v, pos1_hbm.at[pl.ds(base, TPW)])
    for cp in pend[-4:]:
        cp.wait()


@functools.cache
def _b2():
    return functools.partial(
        pl.kernel,
        out_type=[
            jax.ShapeDtypeStruct((NP, HIDDEN), jnp.float32),
            jax.ShapeDtypeStruct((T,), jnp.int32),
            jax.ShapeDtypeStruct((T,), jnp.int32),
            jax.ShapeDtypeStruct((NBP,), jnp.int32),
            jax.ShapeDtypeStruct((16,), jnp.int32),
        ],
        mesh=_mesh(),
        compiler_params=pltpu.CompilerParams(needs_layout_passes=False),
        scratch_types=[
            pltpu.VMEM((NW, 16), jnp.int32),
            pltpu.VMEM((TPW,), jnp.int32),
            pltpu.VMEM((TPW,), jnp.int32),
            pltpu.VMEM((TPW,), jnp.int32),
            pltpu.VMEM((TPW,), jnp.int32),
            pltpu.VMEM((NBP,), jnp.int32),
            pltpu.VMEM((16,), jnp.int32),
            pltpu.VMEM((TPW, HIDDEN), jnp.float32),
            pltpu.VMEM((8, 16), jnp.int32),
            pltpu.SemaphoreType.DMA((4,)),
            pltpu.SemaphoreType.DMA,
        ],
    )(_b2_body)


def _stage_c_body(bexp_ref, nbu_ref, xs_ref, wg_ref, wu_ref, wd_ref, ys_ref,
                  wg_bf, wu_bf, wd_bf):
    i = pl.program_id(0)
    active = i < nbu_ref[0]
    prev = bexp_ref[jnp.maximum(i - 1, 0)]
    changed = jnp.logical_and(active,
                              jnp.logical_or(i == 0, bexp_ref[i] != prev))

    @pl.when(changed)
    def _():
        wg_bf[:] = wg_ref[0].astype(jnp.bfloat16)
        wu_bf[:] = wu_ref[0].astype(jnp.bfloat16)
        wd_bf[:] = wd_ref[0].astype(jnp.bfloat16)

    @pl.when(active)
    def _():
        xb = xs_ref[:].astype(jnp.bfloat16)
        g = jnp.dot(xb, wg_bf[:], preferred_element_type=jnp.float32)
        u = jnp.dot(xb, wu_bf[:], preferred_element_type=jnp.float32)
        h = (_silu(g) * u).astype(jnp.bfloat16)
        ys_ref[:] = jnp.dot(h, wd_bf[:], preferred_element_type=jnp.float32)


def _stage_c(bexp, nbu, xs, w_gate, w_up, w_down):
    grid_spec = pltpu.PrefetchScalarGridSpec(
        num_scalar_prefetch=2,
        grid=(NB,),
        in_specs=[
            pl.BlockSpec((BT, HIDDEN),
                         lambda i, be, nb: (jnp.minimum(i, nb[0] - 1), 0)),
            pl.BlockSpec((1, HIDDEN, D_FF), lambda i, be, nb: (be[i], 0, 0)),
            pl.BlockSpec((1, HIDDEN, D_FF), lambda i, be, nb: (be[i], 0, 0)),
            pl.BlockSpec((1, D_FF, HIDDEN), lambda i, be, nb: (be[i], 0, 0)),
        ],
        out_specs=pl.BlockSpec((BT, HIDDEN), lambda i, be, nb: (i, 0)),
        scratch_shapes=[
            pltpu.VMEM((HIDDEN, D_FF), jnp.bfloat16),
            pltpu.VMEM((HIDDEN, D_FF), jnp.bfloat16),
            pltpu.VMEM((D_FF, HIDDEN), jnp.bfloat16),
        ],
    )
    return pl.pallas_call(
        _stage_c_body,
        grid_spec=grid_spec,
        out_shape=jax.ShapeDtypeStruct((NP, HIDDEN), jnp.float32),
        compiler_params=pltpu.CompilerParams(
            dimension_semantics=("arbitrary",)),
    )(bexp, nbu, xs, w_gate, w_up, w_down)


def _d_body(ys_hbm, pos0_hbm, pos1_hbm, w0_hbm, w1_hbm, sh_hbm, out_hbm,
            p0_v, p1_v, w0_v, w1_v, pc_v, r_v, sh_v, o_v, sems):
    wid = lax.axis_index("s") * 2 + lax.axis_index("c")
    base = wid * TPW
    pltpu.sync_copy(pos0_hbm.at[pl.ds(base, TPW)], p0_v)
    pltpu.sync_copy(pos1_hbm.at[pl.ds(base, TPW)], p1_v)
    pltpu.sync_copy(w0_hbm.at[pl.ds(base, TPW)], w0_v)
    pltpu.sync_copy(w1_hbm.at[pl.ds(base, TPW)], w1_v)
    nch = TPW // 16

    def start(c):
        sl = c & 1
        pc_v[sl, pl.ds(0, 16)] = p0_v[pl.ds(c * 16, 16)]
        pc_v[sl, pl.ds(16, 16)] = p1_v[pl.ds(c * 16, 16)]
        g0 = pltpu.async_copy(ys_hbm.at[pc_v.at[sl, pl.ds(0, 16)]],
                              r_v.at[sl, pl.ds(0, 16)], sems.at[2 * sl])
        g1 = pltpu.async_copy(ys_hbm.at[pc_v.at[sl, pl.ds(16, 16)]],
                              r_v.at[sl, pl.ds(16, 16)], sems.at[2 * sl])
        gs = pltpu.async_copy(sh_hbm.at[pl.ds(base + c * 16, 16)],
                              sh_v.at[sl], sems.at[2 * sl + 1])
        return g0, g1, gs

    pend = start(0)
    for c in range(nch):
        sl = c & 1
        for cp in pend:
            cp.wait()
        if c + 1 < nch:
            pend = start(c + 1)
        w0c = w0_v[pl.ds(c * 16, 16)]
        w1c = w1_v[pl.ds(c * 16, 16)]

        def row_body(r, _):
            w0r = _splat(w0c, r)
            w1r = _splat(w1c, r)
            for j in range(HIDDEN // 16):
                jsl = pl.ds(j * 16, 16)
                o_v[r, jsl] = (w0r * r_v[sl, r, jsl]
                               + w1r * r_v[sl, 16 + r, jsl]
                               + sh_v[sl, r, jsl])
            return 0

        lax.fori_loop(0, 16, row_body, 0)
        pltpu.sync_copy(o_v, out_hbm.at[pl.ds(base + c * 16, 16)])


@functools.cache
def _stage_d():
    return functools.partial(
        pl.kernel,
        out_type=jax.ShapeDtypeStruct((T, HIDDEN), jnp.float32),
        mesh=_mesh(),
        compiler_params=pltpu.CompilerParams(needs_layout_passes=False),
        scratch_types=[
            pltpu.VMEM((TPW,), jnp.int32),
            pltpu.VMEM((TPW,), jnp.int32),
            pltpu.VMEM((TPW,), jnp.float32),
            pltpu.VMEM((TPW,), jnp.float32),
            pltpu.VMEM((2, 32), jnp.int32),
            pltpu.VMEM((2, 32, HIDDEN), jnp.float32),
            pltpu.VMEM((2, 16, HIDDEN), jnp.float32),
            pltpu.VMEM((16, HIDDEN), jnp.float32),
            pltpu.SemaphoreType.DMA((4,)),
        ],
    )(_d_body)


def kernel(hidden_states, gate_w, w_gate, w_up, w_down, sw_gate, sw_up,
           sw_down):
    lt = _stage_a1(hidden_states, gate_w)
    shared = _stage_a2(hidden_states, sw_gate, sw_up, sw_down)
    idx0, idx1, w0, w1, counts = _b1()(lt)
    xs, pos0, pos1, bexp, nbu = _b2()(counts, idx0, idx1, hidden_states)
    ys = _stage_c(bexp, nbu, xs, w_gate, w_up, w_down)
    return _stage_d()(ys, pos0, pos1, w0, w1, shared)


# speedup vs baseline: 1.0231x; 1.0231x over previous
import functools
import jax
import jax.numpy as jnp
from jax import lax
from jax.experimental import pallas as pl
from jax.experimental.pallas import tpu as pltpu
from jax.experimental.pallas import tpu_sc as plsc

HIDDEN = 1024
E = 8
TOP_K = 2
D_FF = 512
N_GROUP = 4
TOPK_GROUP = 2
ROUTED_SCALE = 2.5
T = 2048
SHARED_FF = 1024

NW = 32
TPW = T // NW
BT = 256
LOG_BT = 8
NB = 4096 // BT + 8
NBP = 32
NP = 4096 + 8 * BT
TBA = 512

@functools.cache
def _mesh():
    return plsc.VectorSubcoreMesh(core_axis_name="c", subcore_axis_name="s",
                                  num_cores=2, num_subcores=16)


def _silu(v):
    return v * (1.0 / (1.0 + jnp.exp(-v)))


def _a1_body(x_ref, gate_ref, lt_ref):
    lt_ref[:] = lax.dot_general(gate_ref[:], x_ref[:], (((0,), (1,)), ((), ())),
                                preferred_element_type=jnp.float32)


def _stage_a1(x, gate_w):
    return pl.pallas_call(
        _a1_body,
        grid=(T // TBA,),
        in_specs=[
            pl.BlockSpec((TBA, HIDDEN), lambda i: (i, 0)),
            pl.BlockSpec((HIDDEN, E), lambda i: (0, 0)),
        ],
        out_specs=pl.BlockSpec((E, TBA), lambda i: (0, i)),
        out_shape=jax.ShapeDtypeStruct((E, T), jnp.float32),
        compiler_params=pltpu.CompilerParams(
            dimension_semantics=("arbitrary",)),
    )(x, gate_w)


def _a2_body(x_ref, swg_ref, swu_ref, swd_ref, sh_ref, swg_bf, swu_bf,
             swd_bf):
    @pl.when(pl.program_id(0) == 0)
    def _():
        swg_bf[:] = swg_ref[:].astype(jnp.bfloat16)
        swu_bf[:] = swu_ref[:].astype(jnp.bfloat16)
        swd_bf[:] = swd_ref[:].astype(jnp.bfloat16)

    xbf = x_ref[:].astype(jnp.bfloat16)
    sg = jnp.dot(xbf, swg_bf[:], preferred_element_type=jnp.float32)
    su = jnp.dot(xbf, swu_bf[:], preferred_element_type=jnp.float32)
    hs = (_silu(sg) * su).astype(jnp.bfloat16)
    sh_ref[:] = jnp.dot(hs, swd_bf[:], preferred_element_type=jnp.float32)


def _stage_a2(x, sw_gate, sw_up, sw_down):
    full = lambda shape: pl.BlockSpec(shape, lambda i: (0,) * len(shape))
    return pl.pallas_call(
        _a2_body,
        grid=(T // TBA,),
        in_specs=[
            pl.BlockSpec((TBA, HIDDEN), lambda i: (i, 0)),
            full((HIDDEN, SHARED_FF)),
            full((HIDDEN, SHARED_FF)),
            full((SHARED_FF, HIDDEN)),
        ],
        out_specs=pl.BlockSpec((TBA, HIDDEN), lambda i: (i, 0)),
        out_shape=jax.ShapeDtypeStruct((T, HIDDEN), jnp.float32),
        scratch_shapes=[
            pltpu.VMEM((HIDDEN, SHARED_FF), jnp.bfloat16),
            pltpu.VMEM((HIDDEN, SHARED_FF), jnp.bfloat16),
            pltpu.VMEM((SHARED_FF, HIDDEN), jnp.bfloat16),
        ],
        compiler_params=pltpu.CompilerParams(
            dimension_semantics=("arbitrary",)),
    )(x, sw_gate, sw_up, sw_down)


def _routing_batch(le):
    one = jnp.ones((16,), jnp.int32)
    zi = jnp.zeros((16,), jnp.int32)
    zf = jnp.zeros((16,), jnp.float32)
    gc = [jnp.maximum(le[2 * j], le[2 * j + 1]) for j in range(N_GROUP)]
    gsel = []
    for j in range(N_GROUP):
        beaten = zi
        for j2 in range(N_GROUP):
            if j2 == j:
                continue
            b = (gc[j2] >= gc[j]) if j2 < j else (gc[j2] > gc[j])
            beaten = beaten + jnp.where(b, one, zi)
        gsel.append(beaten < TOPK_GROUP)
    valid = [gsel[e // 2] for e in range(E)]
    esel = []
    for e in range(E):
        beaten = zi
        for e2 in range(E):
            if e2 == e:
                continue
            b = (le[e2] >= le[e]) if e2 < e else (le[e2] > le[e])
            beaten = beaten + jnp.where(valid[e2] & b, one, zi)
        esel.append(valid[e] & (beaten < TOP_K))
    big = jnp.full((16,), 99, jnp.int32)
    neg1 = jnp.full((16,), -1, jnp.int32)
    ev = [jnp.full((16,), e, jnp.int32) for e in range(E)]
    idx0 = functools.reduce(jnp.minimum,
                            [jnp.where(esel[e], ev[e], big) for e in range(E)])
    idx1 = functools.reduce(jnp.maximum,
                            [jnp.where(esel[e], ev[e], neg1) for e in range(E)])
    neginf = jnp.full((16,), -1e30, jnp.float32)
    neg20 = jnp.full((16,), -20.0, jnp.float32)
    m = functools.reduce(
        jnp.maximum, [jnp.where(esel[e], le[e], neginf) for e in range(E)])
    we = [jnp.where(esel[e], jnp.exp(jnp.where(esel[e], le[e] - m, neg20)),
                    zf) for e in range(E)]
    wsum = functools.reduce(jnp.add, we)
    winv = jnp.full((16,), ROUTED_SCALE, jnp.float32) / wsum
    w0 = functools.reduce(
        jnp.add,
        [jnp.where(idx0 == ev[e], we[e], zf) for e in range(E)]) * winv
    w1 = functools.reduce(
        jnp.add,
        [jnp.where(idx1 == ev[e], we[e], zf) for e in range(E)]) * winv
    return idx0, idx1, w0, w1, esel


def _b1_body(lt_hbm, idx0_hbm, idx1_hbm, w0_hbm, w1_hbm, counts_hbm,
             lt_v, i0_v, i1_v, w0_v, w1_v, cnt_v):
    wid = lax.axis_index("s") * 2 + lax.axis_index("c")
    base = wid * TPW
    for e in range(E):
        pltpu.sync_copy(lt_hbm.at[e, pl.ds(base, TPW)], lt_v.at[e])
    ii = lax.broadcasted_iota(jnp.int32, (16,), 0)
    zi = jnp.zeros((16,), jnp.int32)
    cnt = zi
    for b in range(TPW // 16):
        le = [lt_v[e, pl.ds(b * 16, 16)] for e in range(E)]
        idx0, idx1, w0, w1, esel = _routing_batch(le)
        i0_v[pl.ds(b * 16, 16)] = idx0
        i1_v[pl.ds(b * 16, 16)] = idx1
        w0_v[pl.ds(b * 16, 16)] = w0
        w1_v[pl.ds(b * 16, 16)] = w1
        for e in range(E):
            pope = plsc.all_reduce_population_count(esel[e])
            cnt = cnt + jnp.where(ii == jnp.full((16,), e, jnp.int32),
                                  pope, zi)
    cnt_v[...] = cnt
    pltpu.sync_copy(i0_v, idx0_hbm.at[pl.ds(base, TPW)])
    pltpu.sync_copy(i1_v, idx1_hbm.at[pl.ds(base, TPW)])
    pltpu.sync_copy(w0_v, w0_hbm.at[pl.ds(base, TPW)])
    pltpu.sync_copy(w1_v, w1_hbm.at[pl.ds(base, TPW)])
    pltpu.sync_copy(cnt_v, counts_hbm.at[wid])


@functools.cache
def _b1():
    return functools.partial(
        pl.kernel,
        out_type=[
            jax.ShapeDtypeStruct((T,), jnp.int32),
            jax.ShapeDtypeStruct((T,), jnp.int32),
            jax.ShapeDtypeStruct((T,), jnp.float32),
            jax.ShapeDtypeStruct((T,), jnp.float32),
            jax.ShapeDtypeStruct((NW, 16), jnp.int32),
        ],
        mesh=_mesh(),
        compiler_params=pltpu.CompilerParams(needs_layout_passes=False),
        scratch_types=[
            pltpu.VMEM((E, TPW), jnp.float32),
            pltpu.VMEM((TPW,), jnp.int32),
            pltpu.VMEM((TPW,), jnp.int32),
            pltpu.VMEM((TPW,), jnp.float32),
            pltpu.VMEM((TPW,), jnp.float32),
            pltpu.VMEM((16,), jnp.int32),
        ],
    )(_b1_body)


def _splat(v, e_scalar):
    ii = lax.broadcasted_iota(jnp.int32, (16,), 0)
    s = lax.reduce_sum_p.bind(
        jnp.where(ii == jnp.full((16,), e_scalar, jnp.int32), v,
                  jnp.zeros_like(v)), axes=(0,))
    return jnp.full((16,), s, v.dtype)


def _b2_body(counts_hbm, idx0_hbm, idx1_hbm, x_hbm,
             xs_hbm, pos0_hbm, pos1_hbm, bexp_hbm, nbu_hbm,
             counts_v, i0_v, i1_v, p0_v, p1_v, bexp_v, nbu_v, xrows_v,
             pmat_v, sems, semx):
    wid = lax.axis_index("s") * 2 + lax.axis_index("c")
    base = wid * TPW
    ii = lax.broadcasted_iota(jnp.int32, (16,), 0)
    xcopy = pltpu.async_copy(x_hbm.at[pl.ds(base, TPW)], xrows_v, semx)
    pltpu.sync_copy(counts_hbm, counts_v)

    zero = jnp.zeros((16,), jnp.int32)
    wid_v = jnp.full((16,), wid, jnp.int32)
    total = zero
    pre = zero
    for w in range(NW):
        row = counts_v[w]
        total = total + row
        pre = pre + jnp.where(jnp.full((16,), w, jnp.int32) < wid_v, row,
                              zero)
    padded = lax.shift_left(
        lax.shift_right_logical(total + (BT - 1), LOG_BT), LOG_BT)
    cums = plsc.cumsum(padded)
    offp = cums - padded
    wb = offp + pre

    nbu = _splat(cums, 7)
    nbu_blocks = lax.shift_right_logical(nbu, LOG_BT)
    starts = lax.shift_right_logical(offp, LOG_BT)
    nblk = lax.shift_right_logical(padded, LOG_BT)
    laste = lax.reduce_max_p.bind(jnp.where(nblk > zero, ii, zero),
                                  axes=(0,))
    laste_v = jnp.full((16,), laste, jnp.int32)
    for v in range(NBP // 16):
        b = ii + jnp.full((16,), v * 16, jnp.int32)
        val = zero
        for e in range(E):
            s_e = _splat(starts, e)
            n_e = _splat(nblk, e)
            val = val + jnp.where((b >= s_e) & (b < s_e + n_e),
                                  jnp.full((16,), e, jnp.int32), zero)
        val = jnp.where(b >= nbu_blocks, laste_v, val)
        bexp_v[pl.ds(v * 16, 16)] = val
    nbu_v[...] = nbu_blocks

    @pl.when(wid == 0)
    def _():
        pltpu.sync_copy(bexp_v, bexp_hbm)
        pltpu.sync_copy(nbu_v, nbu_hbm)

    pltpu.sync_copy(idx0_hbm.at[pl.ds(base, TPW)], i0_v)
    pltpu.sync_copy(idx1_hbm.at[pl.ds(base, TPW)], i1_v)
    ctr = zero
    onev = jnp.ones((16,), jnp.int32)
    pend = []
    for b in range(TPW // 16):
        for k, (iv, pv) in enumerate(((i0_v, p0_v), (i1_v, p1_v))):
            idxv = iv[pl.ds(b * 16, 16)]
            posv = zero
            for e in range(E):
                e_v = jnp.full((16,), e, jnp.int32)
                msk = idxv == e_v
                mi = jnp.where(msk, onev, zero)
                cs = plsc.cumsum(mi)
                lane_base = _splat(wb + ctr, e)
                posv = posv + jnp.where(msk, lane_base + cs - onev, zero)
                pope = plsc.all_reduce_population_count(msk)
                ctr = ctr + jnp.where(ii == e_v, pope, zero)
            pv[pl.ds(b * 16, 16)] = posv
            j = 2 * b + k
            pmat_v[j] = posv
            if j == 0:
                xcopy.wait()
            if j >= 4:
                pend[j - 4].wait()
            pend.append(pltpu.async_copy(
                xrows_v.at[pl.ds(b * 16, 16)],
                xs_hbm.at[pmat_v.at[j]], sems.at[j % 4]))
    pltpu.sync_copy(p0_v, pos0_hbm.at[pl.ds(base, TPW)])
    pltpu.sync_copy(p1_v, pos1_hbm.at[pl.ds(base, TPW)])
    for cp in pend[-4:]:
        cp.wait()


@functools.cache
def _b2():
    return functools.partial(
        pl.kernel,
        out_type=[
            jax.ShapeDtypeStruct((NP, HIDDEN), jnp.float32),
            jax.ShapeDtypeStruct((T,), jnp.int32),
            jax.ShapeDtypeStruct((T,), jnp.int32),
            jax.ShapeDtypeStruct((NBP,), jnp.int32),
            jax.ShapeDtypeStruct((16,), jnp.int32),
        ],
        mesh=_mesh(),
        compiler_params=pltpu.CompilerParams(needs_layout_passes=False),
        scratch_types=[
            pltpu.VMEM((NW, 16), jnp.int32),
            pltpu.VMEM((TPW,), jnp.int32),
            pltpu.VMEM((TPW,), jnp.int32),
            pltpu.VMEM((TPW,), jnp.int32),
            pltpu.VMEM((TPW,), jnp.int32),
            pltpu.VMEM((NBP,), jnp.int32),
            pltpu.VMEM((16,), jnp.int32),
            pltpu.VMEM((TPW, HIDDEN), jnp.float32),
            pltpu.VMEM((8, 16), jnp.int32),
            pltpu.SemaphoreType.DMA((4,)),
            pltpu.SemaphoreType.DMA,
        ],
    )(_b2_body)


def _stage_c_body(bexp_ref, nbu_ref, xs_ref, wg_ref, wu_ref, wd_ref, ys_ref,
                  wg_bf, wu_bf, wd_bf):
    i = pl.program_id(0)
    active = i < nbu_ref[0]
    prev = bexp_ref[jnp.maximum(i - 1, 0)]
    changed = jnp.logical_and(active,
                              jnp.logical_or(i == 0, bexp_ref[i] != prev))

    @pl.when(changed)
    def _():
        wg_bf[:] = wg_ref[0].astype(jnp.bfloat16)
        wu_bf[:] = wu_ref[0].astype(jnp.bfloat16)
        wd_bf[:] = wd_ref[0].astype(jnp.bfloat16)

    @pl.when(active)
    def _():
        xb = xs_ref[:].astype(jnp.bfloat16)
        g = jnp.dot(xb, wg_bf[:], preferred_element_type=jnp.float32)
        u = jnp.dot(xb, wu_bf[:], preferred_element_type=jnp.float32)
        h = (_silu(g) * u).astype(jnp.bfloat16)
        ys_ref[:] = jnp.dot(h, wd_bf[:], preferred_element_type=jnp.float32)


def _stage_c(bexp, nbu, xs, w_gate, w_up, w_down):
    grid_spec = pltpu.PrefetchScalarGridSpec(
        num_scalar_prefetch=2,
        grid=(NB,),
        in_specs=[
            pl.BlockSpec((BT, HIDDEN),
                         lambda i, be, nb: (jnp.minimum(i, nb[0] - 1), 0)),
            pl.BlockSpec((1, HIDDEN, D_FF), lambda i, be, nb: (be[i], 0, 0)),
            pl.BlockSpec((1, HIDDEN, D_FF), lambda i, be, nb: (be[i], 0, 0)),
            pl.BlockSpec((1, D_FF, HIDDEN), lambda i, be, nb: (be[i], 0, 0)),
        ],
        out_specs=pl.BlockSpec((BT, HIDDEN), lambda i, be, nb: (i, 0)),
        scratch_shapes=[
            pltpu.VMEM((HIDDEN, D_FF), jnp.bfloat16),
            pltpu.VMEM((HIDDEN, D_FF), jnp.bfloat16),
            pltpu.VMEM((D_FF, HIDDEN), jnp.bfloat16),
        ],
    )
    return pl.pallas_call(
        _stage_c_body,
        grid_spec=grid_spec,
        out_shape=jax.ShapeDtypeStruct((NP, HIDDEN), jnp.float32),
        compiler_params=pltpu.CompilerParams(
            dimension_semantics=("arbitrary",)),
    )(bexp, nbu, xs, w_gate, w_up, w_down)


def _d_body(ys_hbm, pos0_hbm, pos1_hbm, w0_hbm, w1_hbm, sh_hbm, out_hbm,
            p0_v, p1_v, w0_v, w1_v, p0c_v, p1c_v, r0_v, r1_v, sh_v, o_v,
            sems):
    wid = lax.axis_index("s") * 2 + lax.axis_index("c")
    base = wid * TPW
    pltpu.sync_copy(pos0_hbm.at[pl.ds(base, TPW)], p0_v)
    pltpu.sync_copy(pos1_hbm.at[pl.ds(base, TPW)], p1_v)
    pltpu.sync_copy(w0_hbm.at[pl.ds(base, TPW)], w0_v)
    pltpu.sync_copy(w1_hbm.at[pl.ds(base, TPW)], w1_v)
    nch = TPW // 16

    def start(c):
        sl = c & 1
        p0c_v[sl] = p0_v[pl.ds(c * 16, 16)]
        p1c_v[sl] = p1_v[pl.ds(c * 16, 16)]
        g0 = pltpu.async_copy(ys_hbm.at[p0c_v.at[sl]], r0_v.at[sl],
                              sems.at[3 * sl])
        g1 = pltpu.async_copy(ys_hbm.at[p1c_v.at[sl]], r1_v.at[sl],
                              sems.at[3 * sl + 1])
        gs = pltpu.async_copy(sh_hbm.at[pl.ds(base + c * 16, 16)],
                              sh_v.at[sl], sems.at[3 * sl + 2])
        return g0, g1, gs

    pend = start(0)
    for c in range(nch):
        sl = c & 1
        for cp in pend:
            cp.wait()
        if c + 1 < nch:
            pend = start(c + 1)
        w0c = w0_v[pl.ds(c * 16, 16)]
        w1c = w1_v[pl.ds(c * 16, 16)]

        def row_body(r, _):
            w0r = _splat(w0c, r)
            w1r = _splat(w1c, r)
            for j in range(HIDDEN // 16):
                jsl = pl.ds(j * 16, 16)
                o_v[r, jsl] = (w0r * r0_v[sl, r, jsl] + w1r * r1_v[sl, r, jsl]
                               + sh_v[sl, r, jsl])
            return 0

        lax.fori_loop(0, 16, row_body, 0)
        pltpu.sync_copy(o_v, out_hbm.at[pl.ds(base + c * 16, 16)])


@functools.cache
def _stage_d():
    return functools.partial(
        pl.kernel,
        out_type=jax.ShapeDtypeStruct((T, HIDDEN), jnp.float32),
        mesh=_mesh(),
        compiler_params=pltpu.CompilerParams(needs_layout_passes=False),
        scratch_types=[
            pltpu.VMEM((TPW,), jnp.int32),
            pltpu.VMEM((TPW,), jnp.int32),
            pltpu.VMEM((TPW,), jnp.float32),
            pltpu.VMEM((TPW,), jnp.float32),
            pltpu.VMEM((2, 16), jnp.int32),
            pltpu.VMEM((2, 16), jnp.int32),
            pltpu.VMEM((2, 16, HIDDEN), jnp.float32),
            pltpu.VMEM((2, 16, HIDDEN), jnp.float32),
            pltpu.VMEM((2, 16, HIDDEN), jnp.float32),
            pltpu.VMEM((16, HIDDEN), jnp.float32),
            pltpu.SemaphoreType.DMA((6,)),
        ],
    )(_d_body)


def kernel(hidden_states, gate_w, w_gate, w_up, w_down, sw_gate, sw_up,
           sw_down):
    lt = _stage_a1(hidden_states, gate_w)
    shared = _stage_a2(hidden_states, sw_gate, sw_up, sw_down)
    idx0, idx1, w0, w1, counts = _b1()(lt)
    xs, pos0, pos1, bexp, nbu = _b2()(counts, idx0, idx1, hidden_states)
    ys = _stage_c(bexp, nbu, xs, w_gate, w_up, w_down)
    return _stage_d()(ys, pos0, pos1, w0, w1, shared)
